# baseline (speedup 1.0000x reference)
import functools

import jax
import jax.numpy as jnp
from jax.experimental import pallas as pl
from jax.experimental.pallas import tpu as pltpu

S, E, C = 4096, 8, 1024
TILE_S = 512
GROUP = 128


def _fused_kernel(logits_ref, m1_ref, m2_ref, loc1_ref, loc2_ref,
                  laux_ref, combine_ref, acc_ref):
    i = pl.program_id(0)
    j = pl.program_id(1)
    n = pl.num_programs(0)
    nj = pl.num_programs(1)

    lg = logits_ref[...]
    m1 = m1_ref[...]
    m2 = m2_ref[...]

    mx = jnp.max(lg, axis=1, keepdims=True)
    ex = jnp.exp(lg - mx)
    gates = ex / jnp.sum(ex, axis=1, keepdims=True)

    @pl.when(jnp.logical_and(i == 0, j == 0))
    def _():
        acc_ref[...] = jnp.zeros_like(acc_ref)

    @pl.when(j == 0)
    def _():
        acc_ref[0:1, :] += jnp.sum(gates, axis=0, keepdims=True)
        acc_ref[1:2, :] += jnp.sum(m1, axis=0, keepdims=True)

    g1s = jnp.sum(gates * m1, axis=1, keepdims=True)
    g2s = jnp.sum(gates * m2, axis=1, keepdims=True)
    denom = jnp.maximum(g1s + g2s, jnp.finfo(jnp.float32).eps)
    g1 = (g1s / denom) * m1
    g2 = (g2s / denom) * m2

    lane = jax.lax.broadcasted_iota(jnp.int32, (E, 2 * GROUP), 1)
    lane31 = jnp.bitwise_and(lane, GROUP - 1)

    for grp in range(TILE_S // GROUP):
        sl = slice(grp * GROUP, (grp + 1) * GROUP)
        gcomb = jnp.concatenate(
            [jnp.swapaxes(g1[sl, :], 0, 1),
             jnp.swapaxes(g2[sl, :], 0, 1)], axis=1).astype(jnp.bfloat16)
        g_rows = [
            jnp.where(lane31 == t, gcomb, jnp.bfloat16(0.0))
            for t in range(GROUP)
        ]
        gmat = jnp.concatenate(g_rows, axis=0)
        lmat = jnp.concatenate(
            [loc1_ref[sl, :], loc2_ref[sl, :]], axis=0
        ).astype(jnp.bfloat16)
        res = jax.lax.dot_general(
            gmat, lmat, (((1,), (0,)), ((), ())),
            preferred_element_type=jnp.float32)
        combine_ref[pl.ds(grp * GROUP * E, GROUP * E), :] = res

    @pl.when(jnp.logical_and(i == n - 1, j == nj - 1))
    def _():
        me_ce = acc_ref[0:1, :] * acc_ref[1:2, :]
        scale = jnp.float32(E) / jnp.float32(S * S)
        laux_ref[...] = jnp.sum(me_ce, axis=1, keepdims=True) * scale


@functools.partial(jax.jit, static_argnames=("interpret",))
def kernel(logits, mask1_float, mask2_float, locations1_sc, locations2_sc,
           interpret=False):
    grid = (S // TILE_S, 2)
    laux, combine2d = pl.pallas_call(
        _fused_kernel,
        grid=grid,
        in_specs=[
            pl.BlockSpec((TILE_S, E), lambda i, j: (i, 0)),
            pl.BlockSpec((TILE_S, E), lambda i, j: (i, 0)),
            pl.BlockSpec((TILE_S, E), lambda i, j: (i, 0)),
            pl.BlockSpec((TILE_S, C // 2), lambda i, j: (i, j)),
            pl.BlockSpec((TILE_S, C // 2), lambda i, j: (i, j)),
        ],
        out_specs=[
            pl.BlockSpec((1, 1), lambda i, j: (0, 0)),
            pl.BlockSpec((TILE_S * E, C // 2), lambda i, j: (i, j)),
        ],
        out_shape=[
            jax.ShapeDtypeStruct((1, 1), jnp.float32),
            jax.ShapeDtypeStruct((S * E, C), jnp.float32),
        ],
        scratch_shapes=[pltpu.VMEM((2, E), jnp.float32)],
        compiler_params=pltpu.CompilerParams(
            dimension_semantics=("arbitrary", "arbitrary"),
        ),
        interpret=interpret,
    )(logits, mask1_float, mask2_float, locations1_sc, locations2_sc)
    return laux[0, 0], combine2d.reshape(S, E, C)


# speedup vs baseline: 1.0361x; 1.0361x over previous
import functools

import jax
import jax.numpy as jnp
from jax.experimental import pallas as pl
from jax.experimental.pallas import tpu as pltpu

S, E, C = 4096, 8, 1024
TILE_S = 512
GROUP = 128


def _fused_kernel(logits_ref, m1_ref, m2_ref, loc1_ref, loc2_ref,
                  laux_ref, combine_ref, acc_ref):
    i = pl.program_id(0)
    n = pl.num_programs(0)

    lg = logits_ref[...]
    m1 = m1_ref[...]
    m2 = m2_ref[...]

    mx = jnp.max(lg, axis=1, keepdims=True)
    ex = jnp.exp(lg - mx)
    gates = ex / jnp.sum(ex, axis=1, keepdims=True)

    @pl.when(i == 0)
    def _():
        acc_ref[...] = jnp.zeros_like(acc_ref)

    acc_ref[0:1, :] += jnp.sum(gates, axis=0, keepdims=True)
    acc_ref[1:2, :] += jnp.sum(m1, axis=0, keepdims=True)

    g1s = jnp.sum(gates * m1, axis=1, keepdims=True)
    g2s = jnp.sum(gates * m2, axis=1, keepdims=True)
    denom = jnp.maximum(g1s + g2s, jnp.finfo(jnp.float32).eps)
    g1 = (g1s / denom) * m1
    g2 = (g2s / denom) * m2

    lane = jax.lax.broadcasted_iota(jnp.int32, (E, 2 * GROUP), 1)
    lane31 = jnp.bitwise_and(lane, GROUP - 1)

    for grp in range(TILE_S // GROUP):
        sl = slice(grp * GROUP, (grp + 1) * GROUP)
        gcomb = jnp.concatenate(
            [jnp.swapaxes(g1[sl, :], 0, 1),
             jnp.swapaxes(g2[sl, :], 0, 1)], axis=1).astype(jnp.bfloat16)
        g_rows = [
            jnp.where(lane31 == t, gcomb, jnp.bfloat16(0.0))
            for t in range(GROUP)
        ]
        gmat = jnp.concatenate(g_rows, axis=0)
        lmat = jnp.concatenate(
            [loc1_ref[sl, :], loc2_ref[sl, :]], axis=0
        ).astype(jnp.bfloat16)
        res = jax.lax.dot_general(
            gmat, lmat, (((1,), (0,)), ((), ())),
            preferred_element_type=jnp.float32)
        combine_ref[pl.ds(grp * GROUP * E, GROUP * E), :] = res

    @pl.when(i == n - 1)
    def _():
        me_ce = acc_ref[0:1, :] * acc_ref[1:2, :]
        scale = jnp.float32(E) / jnp.float32(S * S)
        laux_ref[...] = jnp.sum(me_ce, axis=1, keepdims=True) * scale


@functools.partial(jax.jit, static_argnames=("interpret",))
def kernel(logits, mask1_float, mask2_float, locations1_sc, locations2_sc,
           interpret=False):
    grid = (S // TILE_S,)
    laux, combine2d = pl.pallas_call(
        _fused_kernel,
        grid=grid,
        in_specs=[
            pl.BlockSpec((TILE_S, E), lambda i: (i, 0)),
            pl.BlockSpec((TILE_S, E), lambda i: (i, 0)),
            pl.BlockSpec((TILE_S, E), lambda i: (i, 0)),
            pl.BlockSpec((TILE_S, C), lambda i: (i, 0)),
            pl.BlockSpec((TILE_S, C), lambda i: (i, 0)),
        ],
        out_specs=[
            pl.BlockSpec((1, 1), lambda i: (0, 0)),
            pl.BlockSpec((TILE_S * E, C), lambda i: (i, 0)),
        ],
        out_shape=[
            jax.ShapeDtypeStruct((1, 1), jnp.float32),
            jax.ShapeDtypeStruct((S * E, C), jnp.float32),
        ],
        scratch_shapes=[pltpu.VMEM((2, E), jnp.float32)],
        compiler_params=pltpu.CompilerParams(
            dimension_semantics=("arbitrary",),
        ),
        interpret=interpret,
    )(logits, mask1_float, mask2_float, locations1_sc, locations2_sc)
    return laux[0, 0], combine2d.reshape(S, E, C)


# speedup vs baseline: 1.0365x; 1.0003x over previous
import functools

import jax
import jax.numpy as jnp
from jax.experimental import pallas as pl
from jax.experimental.pallas import tpu as pltpu

S, E, C = 4096, 8, 1024
TILE_S = 512
GROUP = 64


def _fused_kernel(logits_ref, m1_ref, m2_ref, loc1_ref, loc2_ref,
                  laux_ref, combine_ref, acc_ref):
    i = pl.program_id(0)
    n = pl.num_programs(0)

    lg = logits_ref[...]
    m1 = m1_ref[...]
    m2 = m2_ref[...]

    mx = jnp.max(lg, axis=1, keepdims=True)
    ex = jnp.exp(lg - mx)
    gates = ex / jnp.sum(ex, axis=1, keepdims=True)

    @pl.when(i == 0)
    def _():
        acc_ref[...] = jnp.zeros_like(acc_ref)

    acc_ref[0:1, :] += jnp.sum(gates, axis=0, keepdims=True)
    acc_ref[1:2, :] += jnp.sum(m1, axis=0, keepdims=True)

    g1s = jnp.sum(gates * m1, axis=1, keepdims=True)
    g2s = jnp.sum(gates * m2, axis=1, keepdims=True)
    denom = jnp.maximum(g1s + g2s, jnp.finfo(jnp.float32).eps)
    g1 = (g1s / denom) * m1
    g2 = (g2s / denom) * m2

    lane = jax.lax.broadcasted_iota(jnp.int32, (E, 2 * GROUP), 1)
    lane_mod = jnp.bitwise_and(lane, GROUP - 1)

    for grp in range(TILE_S // GROUP):
        sl = slice(grp * GROUP, (grp + 1) * GROUP)
        gcomb = jnp.concatenate(
            [jnp.swapaxes(g1[sl, :], 0, 1),
             jnp.swapaxes(g2[sl, :], 0, 1)], axis=1).astype(jnp.bfloat16)
        g_rows = [
            jnp.where(lane_mod == t, gcomb, jnp.bfloat16(0.0))
            for t in range(GROUP)
        ]
        gmat = jnp.concatenate(g_rows, axis=0)
        lmat = jnp.concatenate(
            [loc1_ref[sl, :], loc2_ref[sl, :]], axis=0
        ).astype(jnp.bfloat16)
        res = jax.lax.dot_general(
            gmat, lmat, (((1,), (0,)), ((), ())),
            preferred_element_type=jnp.float32)
        combine_ref[pl.ds(grp * GROUP * E, GROUP * E), :] = res

    @pl.when(i == n - 1)
    def _():
        me_ce = acc_ref[0:1, :] * acc_ref[1:2, :]
        scale = jnp.float32(E) / jnp.float32(S * S)
        laux_ref[...] = jnp.sum(me_ce, axis=1, keepdims=True) * scale


@functools.partial(jax.jit, static_argnames=("interpret",))
def kernel(logits, mask1_float, mask2_float, locations1_sc, locations2_sc,
           interpret=False):
    grid = (S // TILE_S,)
    laux, combine2d = pl.pallas_call(
        _fused_kernel,
        grid=grid,
        in_specs=[
            pl.BlockSpec((TILE_S, E), lambda i: (i, 0)),
            pl.BlockSpec((TILE_S, E), lambda i: (i, 0)),
            pl.BlockSpec((TILE_S, E), lambda i: (i, 0)),
            pl.BlockSpec((TILE_S, C), lambda i: (i, 0)),
            pl.BlockSpec((TILE_S, C), lambda i: (i, 0)),
        ],
        out_specs=[
            pl.BlockSpec((1, 1), lambda i: (0, 0)),
            pl.BlockSpec((TILE_S * E, C), lambda i: (i, 0)),
        ],
        out_shape=[
            jax.ShapeDtypeStruct((1, 1), jnp.float32),
            jax.ShapeDtypeStruct((S * E, C), jnp.float32),
        ],
        scratch_shapes=[pltpu.VMEM((2, E), jnp.float32)],
        compiler_params=pltpu.CompilerParams(
            dimension_semantics=("arbitrary",),
        ),
        interpret=interpret,
    )(logits, mask1_float, mask2_float, locations1_sc, locations2_sc)
    return laux[0, 0], combine2d.reshape(S, E, C)
